# Initial kernel scaffold; baseline (speedup 1.0000x reference)
import jax
import jax.numpy as jnp
from jax.experimental import pallas as pl


def kernel(tc, embedding):
    raise NotImplementedError("write your pallas kernel here")


# speedup vs baseline: 4.7639x; 4.7639x over previous
import functools

import jax
import jax.numpy as jnp
from jax import lax
from jax.experimental import pallas as pl
from jax.experimental.pallas import tpu as pltpu
from jax.experimental.pallas import tpu_sc as plsc

PERIODS = 10000
FREQS = 32
BATCH = 4096
SEQ = 200
D = 2 * FREQS

NC = 2
NS = 16
NW = NC * NS

B_TOTAL = BATCH * SEQ
B_PER_W = B_TOTAL // NW
CHUNK = 800
N_CHUNKS = B_PER_W // CHUNK


def _gather_body(table_hbm, idx_hbm, out_hbm, idx_v, rows_v, sem):
    wid = lax.axis_index("s") * NC + lax.axis_index("c")
    w_base = wid * B_PER_W

    def step(i, carry):
        base = pl.multiple_of(w_base + i * CHUNK, 8)
        pltpu.sync_copy(idx_hbm.at[pl.ds(base, CHUNK)], idx_v)
        pltpu.async_copy(table_hbm.at[idx_v], rows_v, sem).wait()
        pltpu.sync_copy(rows_v, out_hbm.at[pl.ds(base, CHUNK)])
        return carry

    lax.fori_loop(0, N_CHUNKS, step, 0)


@jax.jit
def _expand(tc, embedding):
    idx = tc.reshape(-1).astype(jnp.int32)
    mesh = plsc.VectorSubcoreMesh(core_axis_name="c", subcore_axis_name="s")
    out = pl.kernel(
        _gather_body,
        out_type=jax.ShapeDtypeStruct((B_TOTAL, D), jnp.float32),
        mesh=mesh,
        scratch_types=[
            pltpu.VMEM((CHUNK,), jnp.int32),
            pltpu.VMEM((CHUNK, D), jnp.float32),
            pltpu.SemaphoreType.DMA,
        ],
        compiler_params=pltpu.CompilerParams(use_tc_tiling_on_sc=False),
    )(embedding, idx)
    return out.reshape(BATCH, SEQ, D)


def kernel(tc, embedding):
    return _expand(tc, embedding)


# speedup vs baseline: 4.9464x; 1.0383x over previous
import jax
import jax.numpy as jnp
from jax import lax
from jax.experimental import pallas as pl
from jax.experimental.pallas import tpu as pltpu
from jax.experimental.pallas import tpu_sc as plsc

PERIODS = 10000
FREQS = 32
BATCH = 4096
SEQ = 200
D = 2 * FREQS

NC = 2
NS = 16
NW = NC * NS

B_TOTAL = BATCH * SEQ
B_PER_W = B_TOTAL // NW
CHUNK = 800
N_CHUNKS = B_PER_W // CHUNK


def _gather_body(table_hbm, idx_hbm, out_hbm, idx_v, rows_v,
                 isem0, isem1, gsem0, gsem1, osem0, osem1):
    wid = lax.axis_index("s") * NC + lax.axis_index("c")
    w_base = wid * B_PER_W
    isem = (isem0, isem1)
    gsem = (gsem0, gsem1)
    osem = (osem0, osem1)

    def idx_start(i, b):
        base = pl.multiple_of(w_base + i * CHUNK, 8)
        pltpu.async_copy(idx_hbm.at[pl.ds(base, CHUNK)], idx_v.at[b], isem[b])

    def idx_wait(i, b):
        base = pl.multiple_of(w_base + i * CHUNK, 8)
        pltpu.make_async_copy(
            idx_hbm.at[pl.ds(base, CHUNK)], idx_v.at[b], isem[b]).wait()

    def gather(b):
        pltpu.async_copy(table_hbm.at[idx_v.at[b]], rows_v.at[b],
                         gsem[b]).wait()

    def out_start(i, b):
        base = pl.multiple_of(w_base + i * CHUNK, 8)
        pltpu.async_copy(rows_v.at[b], out_hbm.at[pl.ds(base, CHUNK)], osem[b])

    def out_wait(i, b):
        base = pl.multiple_of(w_base + i * CHUNK, 8)
        pltpu.make_async_copy(
            rows_v.at[b], out_hbm.at[pl.ds(base, CHUNK)], osem[b]).wait()

    idx_start(0, 0)
    idx_start(1, 1)
    for b in range(2):
        idx_wait(b, b)
        gather(b)
        out_start(b, b)
        idx_start(b + 2, b)

    def step(j, carry):
        for b in range(2):
            i = 2 * j + b
            idx_wait(i, b)
            out_wait(i - 2, b)
            gather(b)
            out_start(i, b)
            idx_start(i + 2, b)
        return carry

    lax.fori_loop(1, N_CHUNKS // 2 - 1, step, 0)

    for b in range(2):
        i = N_CHUNKS - 2 + b
        idx_wait(i, b)
        out_wait(i - 2, b)
        gather(b)
        out_start(i, b)
    for b in range(2):
        out_wait(N_CHUNKS - 2 + b, b)


@jax.jit
def _expand(tc, embedding):
    idx = tc.reshape(-1).astype(jnp.int32)
    mesh = plsc.VectorSubcoreMesh(core_axis_name="c", subcore_axis_name="s")
    out = pl.kernel(
        _gather_body,
        out_type=jax.ShapeDtypeStruct((B_TOTAL, D), jnp.float32),
        mesh=mesh,
        scratch_types=[
            pltpu.VMEM((2, CHUNK), jnp.int32),
            pltpu.VMEM((2, CHUNK, D), jnp.float32),
        ] + [pltpu.SemaphoreType.DMA] * 6,
        compiler_params=pltpu.CompilerParams(use_tc_tiling_on_sc=False),
    )(embedding, idx)
    return out.reshape(BATCH, SEQ, D)


def kernel(tc, embedding):
    return _expand(tc, embedding)
